# baseline (speedup 1.0000x reference)
import functools

import jax
import jax.numpy as jnp
from jax import lax
from jax.experimental import pallas as pl
from jax.experimental.pallas import tpu as pltpu
from jax.experimental.pallas import tpu_sc as plsc

B = 16384
F = 64
H = 32
R = 100000
NC = 2
NS = 16
NW = NC * NS
PPW = B // NW
NBLK = (R + 127) // 128
NBLKP = 792
DEPTH = 16
PBW = NBLKP * DEPTH
BPW2 = 25
PBWIN = 32
PBRD = PBWIN * DEPTH
NGMAX = 4
HCAP = NW * DEPTH + 32
OUTP = 128
BDUM = B + 2048
TAILST = (NBLK - 1) * 128


def _iota16():
  return lax.iota(jnp.int32, 16)


def _shuffle(x, idx):
  return lax.gather(
      x, idx[:, None],
      lax.GatherDimensionNumbers(
          offset_dims=(), collapsed_slice_dims=(0,), start_index_map=(0,)),
      slice_sizes=(1,), mode=lax.GatherScatterMode.PROMISE_IN_BOUNDS)


def _k1_body(uids, iids, posbuf, idbuf, stage, counts):
  w = lax.axis_index("s") * NC + lax.axis_index("c")
  iota = _iota16()
  for s, ids_hbm in ((0, uids), (1, iids)):
    pltpu.sync_copy(ids_hbm.at[pl.ds(w * PPW, PPW)], idbuf)

    def init_stage(r, _):
      stage[pl.ds(r * 16, 16)] = jnp.full((16,), -1, jnp.int32)
      return 0
    lax.fori_loop(0, PBW // 16, init_stage, 0)

    def init_counts(r, _):
      counts[pl.ds(r * 16, 16)] = jnp.zeros((16,), jnp.int32)
      return 0
    lax.fori_loop(0, 49, init_counts, 0)

    def scan(v, _):
      ids = idbuf[pl.ds(v * 16, 16)]
      blk = lax.shift_right_logical(ids, 7)
      lane = lax.bitwise_and(ids, 127)
      pos = w * PPW + v * 16 + iota
      entry = pos * 128 + lane
      rank = jnp.zeros((16,), jnp.int32)
      cnt = jnp.zeros((16,), jnp.int32)
      for sh in range(1, 16):
        prev = _shuffle(blk, jnp.maximum(iota - sh, 0))
        nxt = _shuffle(blk, jnp.minimum(iota + sh, 15))
        pvalid = (iota >= sh).astype(jnp.int32)
        nvalid = (iota < 16 - sh).astype(jnp.int32)
        rank = rank + (prev == blk).astype(jnp.int32) * pvalid
        cnt = cnt + (nxt == blk).astype(jnp.int32) * nvalid
      total = rank + cnt + 1
      base = plsc.load_gather(counts, [blk])
      slot = jnp.minimum(base + rank, DEPTH - 1)
      plsc.store_scatter(stage, [blk * DEPTH + slot], entry)
      plsc.store_scatter(counts, [blk], jnp.minimum(base + total, DEPTH))
      return 0
    lax.fori_loop(0, PPW // 16, scan, 0)
    pltpu.sync_copy(stage, posbuf.at[s, w])


@functools.cache
def _k1():
  return pl.kernel(
      _k1_body,
      out_type=jax.ShapeDtypeStruct((2, NW, PBW), jnp.int32),
      mesh=plsc.VectorSubcoreMesh(core_axis_name="c", subcore_axis_name="s"),
      compiler_params=pltpu.CompilerParams(needs_layout_passes=False),
      scratch_types=[
          pltpu.VMEM((PPW,), jnp.int32),
          pltpu.VMEM((PBW,), jnp.int32),
          pltpu.VMEM((784,), jnp.int32),
      ],
  )


def _drain_n(count, stage, out_hbm, sem):
  def one(r, _):
    pltpu.make_async_copy(
        stage.at[0, pl.ds(0, 16)], out_hbm.at[pl.ds(0, 16)], sem).wait()
    return 0
  lax.fori_loop(0, count, one, 0)


def _process_side(jd, active, pb, slabG, slabM, hit, stage, pidx, out_hbm,
                  sem, sr, fprev):
  iota = _iota16()
  _drain_n(fprev, stage, out_hbm, sem)
  jc = jnp.minimum(jd, PBWIN - 1)

  def comp(g, n):
    flat = g * 16 + iota
    e = plsc.load_gather(
        pb, [lax.shift_right_logical(flat, 4),
             jc * DEPTH + lax.bitwise_and(flat, 15)])
    m = e >= 0
    dest = n + jnp.cumsum(m.astype(jnp.int32)) - 1
    plsc.store_scatter(hit, [dest], e, mask=m)
    return n + plsc.all_reduce_population_count(m)[0]

  n = lax.fori_loop(0, NW * DEPTH // 16, comp, 0)
  n = jnp.minimum(n, NGMAX * 16)
  plsc.store_scatter(hit, [n + iota], jnp.full((16,), -1, jnp.int32))
  ng = jnp.minimum(lax.shift_right_logical(n + 15, 4), NGMAX)

  for g in range(NGMAX):
    def one_group():
      ents = hit[pl.ds(g * 16, 16)]
      lanes = lax.bitwise_and(ents, 127)
      pos = jnp.where((ents < 0) | jnp.logical_not(active), B,
                      lax.shift_right_arithmetic(ents, 7))
      pidx[sr, g] = pos
      for f in range(F):
        vals = plsc.load_gather(
            slabG, [jnp.full((16,), f, jnp.int32), lanes])
        plsc.store_scatter(
            stage.at[sr], [g * 16 + iota, jnp.full((16,), f, jnp.int32)],
            vals)
      for f in range(H):
        vals = plsc.load_gather(
            slabM, [jnp.full((16,), f, jnp.int32), lanes])
        plsc.store_scatter(
            stage.at[sr], [g * 16 + iota, jnp.full((16,), F + f, jnp.int32)],
            vals)
      pltpu.async_copy(stage.at[sr, pl.ds(g * 16, 16)],
                       out_hbm.at[pidx.at[sr, g]], sem)
    pl.when(g < ng)(one_group)
  return ng


def _k2_body(guT, giT, muT, miT, posbuf, tGU, tGI, tMU, tMI, outU, outI,
             pbU, pbI, sGU, sGI, sMU, sMI, hitU, hitI,
             stage, pidx, semA, semS):
  w = lax.axis_index("s") * NC + lax.axis_index("c")
  b0 = jnp.minimum(w * 24 + jnp.minimum(w, 14), NBLK - BPW2)
  b0a = (b0 // 8) * 8
  joff = b0 - b0a
  off = pl.multiple_of(b0a * DEPTH, 128)
  pltpu.sync_copy(posbuf.at[0, :, pl.ds(off, PBRD)], pbU)
  pltpu.sync_copy(posbuf.at[1, :, pl.ds(off, PBRD)], pbI)

  def eachslab(fn):
    fn(guT, tGU, sGU)
    fn(giT, tGI, sGI)
    fn(muT, tMU, sMU)
    fn(miT, tMI, sMI)

  def fire(b, buf):
    st = pl.multiple_of(jnp.minimum(b, NBLK - 2) * 128, 128)

    def full():
      eachslab(lambda t, tl, s: pltpu.async_copy(
          t.at[:, pl.ds(st, 128)], s.at[buf], semA))

    def tail():
      eachslab(lambda t, tl, s: pltpu.async_copy(tl, s.at[buf], semA))
    pl.when(b < NBLK - 1)(full)
    pl.when(b >= NBLK - 1)(tail)

  def wait4(b, buf):
    st = pl.multiple_of(jnp.minimum(b, NBLK - 2) * 128, 128)

    def full():
      eachslab(lambda t, tl, s: pltpu.make_async_copy(
          t.at[:, pl.ds(st, 128)], s.at[buf], semA).wait())

    def tail():
      eachslab(lambda t, tl, s: pltpu.make_async_copy(
          tl, s.at[buf], semA).wait())
    pl.when(b < NBLK - 1)(full)
    pl.when(b >= NBLK - 1)(tail)

  fire(b0, 0)

  def pair(p, carry):
    f = list(carry)
    for half in range(2):
      j = p * 2 + half
      active = j < BPW2
      b = b0 + j

      def dma_step():
        def fire_next():
          fire(b + 1, 1 - half)
        pl.when(j < BPW2 - 1)(fire_next)
        wait4(b, half)
      pl.when(active)(dma_step)
      su = 2 * half
      f[su] = _process_side(joff + j, active, pbU, sGU.at[half],
                            sMU.at[half], hitU, stage, pidx, outU, semS,
                            su, f[su])
      f[su + 1] = _process_side(joff + j, active, pbI, sGI.at[half],
                                sMI.at[half], hitI, stage, pidx, outI, semS,
                                su + 1, f[su + 1])
    return tuple(f)

  zero = jnp.int32(0)
  fin = lax.fori_loop(0, (BPW2 + 1) // 2, pair, (zero, zero, zero, zero))
  for k in range(4):
    _drain_n(fin[k], stage, outU, semS)


@functools.cache
def _k2():
  return pl.kernel(
      _k2_body,
      out_type=(
          jax.ShapeDtypeStruct((BDUM, OUTP), jnp.float32),
          jax.ShapeDtypeStruct((BDUM, OUTP), jnp.float32),
      ),
      mesh=plsc.VectorSubcoreMesh(core_axis_name="c", subcore_axis_name="s"),
      compiler_params=pltpu.CompilerParams(needs_layout_passes=False),
      scratch_types=[
          pltpu.VMEM((NW, PBRD), jnp.int32),
          pltpu.VMEM((NW, PBRD), jnp.int32),
          pltpu.VMEM((2, F, 128), jnp.float32),
          pltpu.VMEM((2, F, 128), jnp.float32),
          pltpu.VMEM((2, H, 128), jnp.float32),
          pltpu.VMEM((2, H, 128), jnp.float32),
          pltpu.VMEM((HCAP,), jnp.int32),
          pltpu.VMEM((HCAP,), jnp.int32),
          pltpu.VMEM((4, NGMAX * 16, OUTP), jnp.float32),
          pltpu.VMEM((4, NGMAX, 16), jnp.int32),
          pltpu.SemaphoreType.DMA,
          pltpu.SemaphoreType.DMA,
      ],
  )


def _sigmoid(x):
  return 1.0 / (1.0 + jnp.exp(-x))


BLK = 2048


def _tc_dense_body(outu, outi, w1a, w1b, w2, w3, w4, b1, b2, b3, b4,
                   ow, ob, out):
  pu = outu[...]
  pi = outi[...]
  gu = pu[:, :F]
  mu = pu[:, F:F + H]
  gi = pi[:, :F]
  mi = pi[:, F:F + H]
  gmf = _sigmoid(jnp.sum(gu * gi, axis=1, keepdims=True))
  v = jnp.maximum(
      jnp.dot(mu, w1a[...], preferred_element_type=jnp.float32)
      + jnp.dot(mi, w1b[...], preferred_element_type=jnp.float32)
      + b1[...], 0.0)
  v = jnp.maximum(
      jnp.dot(v, w2[...], preferred_element_type=jnp.float32) + b2[...], 0.0)
  v = jnp.maximum(
      jnp.dot(v, w3[...], preferred_element_type=jnp.float32) + b3[...], 0.0)
  mlp = _sigmoid(jnp.sum(v * w4[...], axis=1, keepdims=True) + b4[...])
  oww = ow[...]
  out[...] = _sigmoid(gmf * oww[0:1, 0:1] + mlp * oww[0:1, 1:2] + ob[...])


def _tc_dense(outu, outi, w1a, w1b, w2, w3, w4, b1, b2, b3, b4, ow, ob):
  full = lambda shape: pl.BlockSpec(shape, lambda i: (0, 0))
  return pl.pallas_call(
      _tc_dense_body,
      grid=(B // BLK,),
      in_specs=[
          pl.BlockSpec((BLK, OUTP), lambda i: (i, 0)),
          pl.BlockSpec((BLK, OUTP), lambda i: (i, 0)),
          full((H, F)),
          full((H, F)),
          full((F, F)),
          full((F, F)),
          full((1, F)),
          full((1, F)),
          full((1, F)),
          full((1, F)),
          full((1, 1)),
          full((1, 2)),
          full((1, 1)),
      ],
      out_specs=pl.BlockSpec((BLK, 1), lambda i: (i, 0)),
      out_shape=jax.ShapeDtypeStruct((B, 1), jnp.float32),
  )(outu, outi, w1a, w1b, w2, w3, w4, b1, b2, b3, b4, ow, ob)


@jax.jit
def kernel(user_ids, item_ids, gmf_user_emb, gmf_item_emb, mlp_user_emb,
           mlp_item_emb, fc_w1, fc_b1, fc_w2, fc_b2, fc_w3, fc_b3,
           mlp_out_w, mlp_out_b, out_w, out_b):
  uids = jnp.asarray(user_ids, jnp.int32)
  iids = jnp.asarray(item_ids, jnp.int32)
  posbuf = _k1()(uids, iids)
  pad = lambda x: jnp.pad(x[TAILST:].T, ((0, 0), (0, 128 - (R - TAILST))))
  outU, outI = _k2()(gmf_user_emb.T, gmf_item_emb.T, mlp_user_emb.T,
                     mlp_item_emb.T, posbuf,
                     pad(gmf_user_emb), pad(gmf_item_emb),
                     pad(mlp_user_emb), pad(mlp_item_emb))
  w1a = fc_w1[:, :H].T
  w1b = fc_w1[:, H:].T
  return _tc_dense(outU, outI, w1a, w1b, fc_w2.T, fc_w3.T,
                   mlp_out_w.reshape(1, F), fc_b1.reshape(1, F),
                   fc_b2.reshape(1, F), fc_b3.reshape(1, F),
                   mlp_out_b.reshape(1, 1), out_w, out_b.reshape(1, 1))


# speedup vs baseline: 1.0065x; 1.0065x over previous
import functools

import jax
import jax.numpy as jnp
from jax import lax
from jax.experimental import pallas as pl
from jax.experimental.pallas import tpu as pltpu
from jax.experimental.pallas import tpu_sc as plsc

B = 16384
F = 64
H = 32
R = 100000
NC = 2
NS = 16
NW = NC * NS
PPW = B // NW
NBLK = (R + 127) // 128
NBLKP = 792
DEPTH = 16
PBW = NBLKP * DEPTH
BPW2 = 25
PBWIN = 32
PBRD = PBWIN * DEPTH
NGMAX = 4
HCAP = NW * DEPTH + 32
OUTP = 128
BDUM = B + 2048
TAILST = (NBLK - 1) * 128


def _iota16():
  return lax.iota(jnp.int32, 16)


def _shuffle(x, idx):
  return lax.gather(
      x, idx[:, None],
      lax.GatherDimensionNumbers(
          offset_dims=(), collapsed_slice_dims=(0,), start_index_map=(0,)),
      slice_sizes=(1,), mode=lax.GatherScatterMode.PROMISE_IN_BOUNDS)


def _k1_body(uids, iids, posbuf, idbuf, stage, counts):
  w = lax.axis_index("s") * NC + lax.axis_index("c")
  iota = _iota16()
  for s, ids_hbm in ((0, uids), (1, iids)):
    pltpu.sync_copy(ids_hbm.at[pl.ds(w * PPW, PPW)], idbuf)

    def init_stage(r, _):
      stage[pl.ds(r * 16, 16)] = jnp.full((16,), -1, jnp.int32)
      return 0
    lax.fori_loop(0, PBW // 16, init_stage, 0)

    def init_counts(r, _):
      counts[pl.ds(r * 16, 16)] = jnp.zeros((16,), jnp.int32)
      return 0
    lax.fori_loop(0, 49, init_counts, 0)

    def scan(v, _):
      ids = idbuf[pl.ds(v * 16, 16)]
      blk = lax.shift_right_logical(ids, 7)
      lane = lax.bitwise_and(ids, 127)
      pos = w * PPW + v * 16 + iota
      entry = pos * 128 + lane
      rank = jnp.zeros((16,), jnp.int32)
      cnt = jnp.zeros((16,), jnp.int32)
      for sh in range(1, 16):
        prev = _shuffle(blk, jnp.maximum(iota - sh, 0))
        nxt = _shuffle(blk, jnp.minimum(iota + sh, 15))
        pvalid = (iota >= sh).astype(jnp.int32)
        nvalid = (iota < 16 - sh).astype(jnp.int32)
        rank = rank + (prev == blk).astype(jnp.int32) * pvalid
        cnt = cnt + (nxt == blk).astype(jnp.int32) * nvalid
      total = rank + cnt + 1
      base = plsc.load_gather(counts, [blk])
      slot = jnp.minimum(base + rank, DEPTH - 1)
      plsc.store_scatter(stage, [blk * DEPTH + slot], entry)
      plsc.store_scatter(counts, [blk], jnp.minimum(base + total, DEPTH))
      return 0
    lax.fori_loop(0, PPW // 16, scan, 0)
    pltpu.sync_copy(stage, posbuf.at[s, w])


@functools.cache
def _k1():
  return pl.kernel(
      _k1_body,
      out_type=jax.ShapeDtypeStruct((2, NW, PBW), jnp.int32),
      mesh=plsc.VectorSubcoreMesh(core_axis_name="c", subcore_axis_name="s"),
      compiler_params=pltpu.CompilerParams(needs_layout_passes=False),
      scratch_types=[
          pltpu.VMEM((PPW,), jnp.int32),
          pltpu.VMEM((PBW,), jnp.int32),
          pltpu.VMEM((784,), jnp.int32),
      ],
  )


def _drain_n(count, stage, out_hbm, sem):
  def one(r, _):
    pltpu.make_async_copy(
        stage.at[0, pl.ds(0, 16)], out_hbm.at[pl.ds(0, 16)], sem).wait()
    return 0
  lax.fori_loop(0, count, one, 0)


def _process_side(jd, active, pb, slabG, slabM, hit, stage, pidx, out_hbm,
                  sem, sr, fprev):
  iota = _iota16()
  _drain_n(fprev, stage, out_hbm, sem)
  jc = jnp.minimum(jd, PBWIN - 1)

  def comp4(q, n):
    es, ms = [], []
    for k in range(4):
      flat = (q * 4 + k) * 16 + iota
      e = plsc.load_gather(
          pb, [lax.shift_right_logical(flat, 4),
               jc * DEPTH + lax.bitwise_and(flat, 15)])
      es.append(e)
      ms.append(e >= 0)
    css = [jnp.cumsum(m.astype(jnp.int32)) for m in ms]
    ps = [plsc.all_reduce_population_count(m)[0] for m in ms]
    off = n
    for k in range(4):
      plsc.store_scatter(hit, [off + css[k] - 1], es[k], mask=ms[k])
      off = off + ps[k]
    return off

  n = lax.fori_loop(0, NW * DEPTH // 16 // 4, comp4, 0)
  n = jnp.minimum(n, NGMAX * 16)
  plsc.store_scatter(hit, [n + iota], jnp.full((16,), -1, jnp.int32))
  ng = jnp.minimum(lax.shift_right_logical(n + 15, 4), NGMAX)

  for g in range(NGMAX):
    def one_group():
      ents = hit[pl.ds(g * 16, 16)]
      lanes = lax.bitwise_and(ents, 127)
      pos = jnp.where((ents < 0) | jnp.logical_not(active), B,
                      lax.shift_right_arithmetic(ents, 7))
      pidx[sr, g] = pos
      for base_f in range(0, F, 8):
        vals = [plsc.load_gather(
            slabG, [jnp.full((16,), base_f + k, jnp.int32), lanes])
            for k in range(8)]
        for k in range(8):
          plsc.store_scatter(
              stage.at[sr],
              [g * 16 + iota, jnp.full((16,), base_f + k, jnp.int32)],
              vals[k])
      for base_f in range(0, H, 8):
        vals = [plsc.load_gather(
            slabM, [jnp.full((16,), base_f + k, jnp.int32), lanes])
            for k in range(8)]
        for k in range(8):
          plsc.store_scatter(
              stage.at[sr],
              [g * 16 + iota, jnp.full((16,), F + base_f + k, jnp.int32)],
              vals[k])
      pltpu.async_copy(stage.at[sr, pl.ds(g * 16, 16)],
                       out_hbm.at[pidx.at[sr, g]], sem)
    pl.when(g < ng)(one_group)
  return ng


def _k2_body(guT, giT, muT, miT, posbuf, tGU, tGI, tMU, tMI, outU, outI,
             pbU, pbI, sGU, sGI, sMU, sMI, hitU, hitI,
             stage, pidx, semA, semS):
  w = lax.axis_index("s") * NC + lax.axis_index("c")
  b0 = jnp.minimum(w * 24 + jnp.minimum(w, 14), NBLK - BPW2)
  b0a = (b0 // 8) * 8
  joff = b0 - b0a
  off = pl.multiple_of(b0a * DEPTH, 128)
  pltpu.sync_copy(posbuf.at[0, :, pl.ds(off, PBRD)], pbU)
  pltpu.sync_copy(posbuf.at[1, :, pl.ds(off, PBRD)], pbI)

  def eachslab(fn):
    fn(guT, tGU, sGU)
    fn(giT, tGI, sGI)
    fn(muT, tMU, sMU)
    fn(miT, tMI, sMI)

  def fire(b, buf):
    st = pl.multiple_of(jnp.minimum(b, NBLK - 2) * 128, 128)

    def full():
      eachslab(lambda t, tl, s: pltpu.async_copy(
          t.at[:, pl.ds(st, 128)], s.at[buf], semA))

    def tail():
      eachslab(lambda t, tl, s: pltpu.async_copy(tl, s.at[buf], semA))
    pl.when(b < NBLK - 1)(full)
    pl.when(b >= NBLK - 1)(tail)

  def wait4(b, buf):
    st = pl.multiple_of(jnp.minimum(b, NBLK - 2) * 128, 128)

    def full():
      eachslab(lambda t, tl, s: pltpu.make_async_copy(
          t.at[:, pl.ds(st, 128)], s.at[buf], semA).wait())

    def tail():
      eachslab(lambda t, tl, s: pltpu.make_async_copy(
          tl, s.at[buf], semA).wait())
    pl.when(b < NBLK - 1)(full)
    pl.when(b >= NBLK - 1)(tail)

  fire(b0, 0)

  def pair(p, carry):
    f = list(carry)
    for half in range(2):
      j = p * 2 + half
      active = j < BPW2
      b = b0 + j

      def dma_step():
        def fire_next():
          fire(b + 1, 1 - half)
        pl.when(j < BPW2 - 1)(fire_next)
        wait4(b, half)
      pl.when(active)(dma_step)
      su = 2 * half
      f[su] = _process_side(joff + j, active, pbU, sGU.at[half],
                            sMU.at[half], hitU, stage, pidx, outU, semS,
                            su, f[su])
      f[su + 1] = _process_side(joff + j, active, pbI, sGI.at[half],
                                sMI.at[half], hitI, stage, pidx, outI, semS,
                                su + 1, f[su + 1])
    return tuple(f)

  zero = jnp.int32(0)
  fin = lax.fori_loop(0, (BPW2 + 1) // 2, pair, (zero, zero, zero, zero))
  for k in range(4):
    _drain_n(fin[k], stage, outU, semS)


@functools.cache
def _k2():
  return pl.kernel(
      _k2_body,
      out_type=(
          jax.ShapeDtypeStruct((BDUM, OUTP), jnp.float32),
          jax.ShapeDtypeStruct((BDUM, OUTP), jnp.float32),
      ),
      mesh=plsc.VectorSubcoreMesh(core_axis_name="c", subcore_axis_name="s"),
      compiler_params=pltpu.CompilerParams(needs_layout_passes=False),
      scratch_types=[
          pltpu.VMEM((NW, PBRD), jnp.int32),
          pltpu.VMEM((NW, PBRD), jnp.int32),
          pltpu.VMEM((2, F, 128), jnp.float32),
          pltpu.VMEM((2, F, 128), jnp.float32),
          pltpu.VMEM((2, H, 128), jnp.float32),
          pltpu.VMEM((2, H, 128), jnp.float32),
          pltpu.VMEM((HCAP,), jnp.int32),
          pltpu.VMEM((HCAP,), jnp.int32),
          pltpu.VMEM((4, NGMAX * 16, OUTP), jnp.float32),
          pltpu.VMEM((4, NGMAX, 16), jnp.int32),
          pltpu.SemaphoreType.DMA,
          pltpu.SemaphoreType.DMA,
      ],
  )


def _sigmoid(x):
  return 1.0 / (1.0 + jnp.exp(-x))


BLK = 2048


def _tc_dense_body(outu, outi, w1a, w1b, w2, w3, w4, b1, b2, b3, b4,
                   ow, ob, out):
  pu = outu[...]
  pi = outi[...]
  gu = pu[:, :F]
  mu = pu[:, F:F + H]
  gi = pi[:, :F]
  mi = pi[:, F:F + H]
  gmf = _sigmoid(jnp.sum(gu * gi, axis=1, keepdims=True))
  v = jnp.maximum(
      jnp.dot(mu, w1a[...], preferred_element_type=jnp.float32)
      + jnp.dot(mi, w1b[...], preferred_element_type=jnp.float32)
      + b1[...], 0.0)
  v = jnp.maximum(
      jnp.dot(v, w2[...], preferred_element_type=jnp.float32) + b2[...], 0.0)
  v = jnp.maximum(
      jnp.dot(v, w3[...], preferred_element_type=jnp.float32) + b3[...], 0.0)
  mlp = _sigmoid(jnp.sum(v * w4[...], axis=1, keepdims=True) + b4[...])
  oww = ow[...]
  out[...] = _sigmoid(gmf * oww[0:1, 0:1] + mlp * oww[0:1, 1:2] + ob[...])


def _tc_dense(outu, outi, w1a, w1b, w2, w3, w4, b1, b2, b3, b4, ow, ob):
  full = lambda shape: pl.BlockSpec(shape, lambda i: (0, 0))
  return pl.pallas_call(
      _tc_dense_body,
      grid=(B // BLK,),
      in_specs=[
          pl.BlockSpec((BLK, OUTP), lambda i: (i, 0)),
          pl.BlockSpec((BLK, OUTP), lambda i: (i, 0)),
          full((H, F)),
          full((H, F)),
          full((F, F)),
          full((F, F)),
          full((1, F)),
          full((1, F)),
          full((1, F)),
          full((1, F)),
          full((1, 1)),
          full((1, 2)),
          full((1, 1)),
      ],
      out_specs=pl.BlockSpec((BLK, 1), lambda i: (i, 0)),
      out_shape=jax.ShapeDtypeStruct((B, 1), jnp.float32),
  )(outu, outi, w1a, w1b, w2, w3, w4, b1, b2, b3, b4, ow, ob)


@jax.jit
def kernel(user_ids, item_ids, gmf_user_emb, gmf_item_emb, mlp_user_emb,
           mlp_item_emb, fc_w1, fc_b1, fc_w2, fc_b2, fc_w3, fc_b3,
           mlp_out_w, mlp_out_b, out_w, out_b):
  uids = jnp.asarray(user_ids, jnp.int32)
  iids = jnp.asarray(item_ids, jnp.int32)
  posbuf = _k1()(uids, iids)
  pad = lambda x: jnp.pad(x[TAILST:].T, ((0, 0), (0, 128 - (R - TAILST))))
  outU, outI = _k2()(gmf_user_emb.T, gmf_item_emb.T, mlp_user_emb.T,
                     mlp_item_emb.T, posbuf,
                     pad(gmf_user_emb), pad(gmf_item_emb),
                     pad(mlp_user_emb), pad(mlp_item_emb))
  w1a = fc_w1[:, :H].T
  w1b = fc_w1[:, H:].T
  return _tc_dense(outU, outI, w1a, w1b, fc_w2.T, fc_w3.T,
                   mlp_out_w.reshape(1, F), fc_b1.reshape(1, F),
                   fc_b2.reshape(1, F), fc_b3.reshape(1, F),
                   mlp_out_b.reshape(1, 1), out_w, out_b.reshape(1, 1))


# speedup vs baseline: 1.0216x; 1.0151x over previous
import functools

import jax
import jax.numpy as jnp
from jax import lax
from jax.experimental import pallas as pl
from jax.experimental.pallas import tpu as pltpu
from jax.experimental.pallas import tpu_sc as plsc

B = 16384
F = 64
H = 32
R = 100000
NC = 2
NS = 16
NW = NC * NS
PPW = B // NW
NBLK = (R + 127) // 128
NBLKP = 792
DEPTH = 16
PBW = NBLKP * DEPTH
BPW2 = 25
PBWIN = 32
PBRD = PBWIN * DEPTH
NGMAX = 4
HCAP = NW * DEPTH + 32
OUTP = 128
BDUM = B + 2048
TAILST = (NBLK - 1) * 128


def _iota16():
  return lax.iota(jnp.int32, 16)


def _shuffle(x, idx):
  return lax.gather(
      x, idx[:, None],
      lax.GatherDimensionNumbers(
          offset_dims=(), collapsed_slice_dims=(0,), start_index_map=(0,)),
      slice_sizes=(1,), mode=lax.GatherScatterMode.PROMISE_IN_BOUNDS)


def _k1_body(uids, iids, posbuf, idbuf, stage, counts):
  w = lax.axis_index("s") * NC + lax.axis_index("c")
  iota = _iota16()
  for s, ids_hbm in ((0, uids), (1, iids)):
    pltpu.sync_copy(ids_hbm.at[pl.ds(w * PPW, PPW)], idbuf)

    def init_stage(r, _):
      stage[pl.ds(r * 16, 16)] = jnp.full((16,), -1, jnp.int32)
      return 0
    lax.fori_loop(0, PBW // 16, init_stage, 0)

    def init_counts(r, _):
      counts[pl.ds(r * 16, 16)] = jnp.zeros((16,), jnp.int32)
      return 0
    lax.fori_loop(0, 49, init_counts, 0)

    def scan(v, _):
      ids = idbuf[pl.ds(v * 16, 16)]
      blk = lax.shift_right_logical(ids, 7)
      lane = lax.bitwise_and(ids, 127)
      pos = w * PPW + v * 16 + iota
      entry = pos * 128 + lane
      rank = jnp.zeros((16,), jnp.int32)
      cnt = jnp.zeros((16,), jnp.int32)
      for sh in range(1, 16):
        prev = _shuffle(blk, jnp.maximum(iota - sh, 0))
        nxt = _shuffle(blk, jnp.minimum(iota + sh, 15))
        pvalid = (iota >= sh).astype(jnp.int32)
        nvalid = (iota < 16 - sh).astype(jnp.int32)
        rank = rank + (prev == blk).astype(jnp.int32) * pvalid
        cnt = cnt + (nxt == blk).astype(jnp.int32) * nvalid
      total = rank + cnt + 1
      base = plsc.load_gather(counts, [blk])
      slot = jnp.minimum(base + rank, DEPTH - 1)
      plsc.store_scatter(stage, [blk * DEPTH + slot], entry)
      plsc.store_scatter(counts, [blk], jnp.minimum(base + total, DEPTH))
      return 0
    lax.fori_loop(0, PPW // 16, scan, 0)
    pltpu.sync_copy(stage, posbuf.at[s, w])


@functools.cache
def _k1():
  return pl.kernel(
      _k1_body,
      out_type=jax.ShapeDtypeStruct((2, NW, PBW), jnp.int32),
      mesh=plsc.VectorSubcoreMesh(core_axis_name="c", subcore_axis_name="s"),
      compiler_params=pltpu.CompilerParams(needs_layout_passes=False),
      scratch_types=[
          pltpu.VMEM((PPW,), jnp.int32),
          pltpu.VMEM((PBW,), jnp.int32),
          pltpu.VMEM((784,), jnp.int32),
      ],
  )


def _drain_n(count, stage, out_hbm, sem):
  def one(r, _):
    pltpu.make_async_copy(
        stage.at[0, pl.ds(0, 16)], out_hbm.at[pl.ds(0, 16)], sem).wait()
    return 0
  lax.fori_loop(0, count, one, 0)


def _process_side(jd, active, pb, slabG, slabM, hit, stage, pidx, out_hbm,
                  sem, sr, fprev):
  iota = _iota16()
  _drain_n(fprev, stage, out_hbm, sem)
  jc = jnp.minimum(jd, PBWIN - 1)

  def comp4(q, n):
    es, ms = [], []
    for k in range(4):
      flat = (q * 4 + k) * 16 + iota
      e = plsc.load_gather(
          pb, [lax.shift_right_logical(flat, 4),
               jc * DEPTH + lax.bitwise_and(flat, 15)])
      es.append(e)
      ms.append(e >= 0)
    css = [jnp.cumsum(m.astype(jnp.int32)) for m in ms]
    ps = [plsc.all_reduce_population_count(m)[0] for m in ms]
    off = n
    for k in range(4):
      plsc.store_scatter(hit, [off + css[k] - 1], es[k], mask=ms[k])
      off = off + ps[k]
    return off

  n = lax.fori_loop(0, NW * DEPTH // 16 // 4, comp4, 0)
  n = jnp.minimum(n, NGMAX * 16)
  plsc.store_scatter(hit, [n + iota], jnp.full((16,), -1, jnp.int32))
  ng = jnp.minimum(lax.shift_right_logical(n + 15, 4), NGMAX)

  for g in range(NGMAX):
    def one_group():
      ents = hit[pl.ds(g * 16, 16)]
      lanes = lax.bitwise_and(ents, 127)
      pos = jnp.where((ents < 0) | jnp.logical_not(active), B,
                      lax.shift_right_arithmetic(ents, 7))
      pidx[sr, g] = pos
      @plsc.parallel_loop(0, F, unroll=8)
      def gmf_feat(f):
        fv = jnp.full((16,), 0, jnp.int32) + f
        vals = plsc.load_gather(slabG, [fv, lanes])
        plsc.store_scatter(stage.at[sr], [g * 16 + iota, fv], vals)

      @plsc.parallel_loop(0, H, unroll=8)
      def mlp_feat(f):
        fv = jnp.full((16,), 0, jnp.int32) + f
        vals = plsc.load_gather(slabM, [fv, lanes])
        plsc.store_scatter(stage.at[sr], [g * 16 + iota, fv + F], vals)
      pltpu.async_copy(stage.at[sr, pl.ds(g * 16, 16)],
                       out_hbm.at[pidx.at[sr, g]], sem)
    pl.when(g < ng)(one_group)
  return ng


def _k2_body(guT, giT, muT, miT, posbuf, tGU, tGI, tMU, tMI, outU, outI,
             pbU, pbI, sGU, sGI, sMU, sMI, hitU, hitI,
             stage, pidx, semA, semS):
  w = lax.axis_index("s") * NC + lax.axis_index("c")
  b0 = jnp.minimum(w * 24 + jnp.minimum(w, 14), NBLK - BPW2)
  b0a = (b0 // 8) * 8
  joff = b0 - b0a
  off = pl.multiple_of(b0a * DEPTH, 128)
  pltpu.sync_copy(posbuf.at[0, :, pl.ds(off, PBRD)], pbU)
  pltpu.sync_copy(posbuf.at[1, :, pl.ds(off, PBRD)], pbI)

  def eachslab(fn):
    fn(guT, tGU, sGU)
    fn(giT, tGI, sGI)
    fn(muT, tMU, sMU)
    fn(miT, tMI, sMI)

  def fire(b, buf):
    st = pl.multiple_of(jnp.minimum(b, NBLK - 2) * 128, 128)

    def full():
      eachslab(lambda t, tl, s: pltpu.async_copy(
          t.at[:, pl.ds(st, 128)], s.at[buf], semA))

    def tail():
      eachslab(lambda t, tl, s: pltpu.async_copy(tl, s.at[buf], semA))
    pl.when(b < NBLK - 1)(full)
    pl.when(b >= NBLK - 1)(tail)

  def wait4(b, buf):
    st = pl.multiple_of(jnp.minimum(b, NBLK - 2) * 128, 128)

    def full():
      eachslab(lambda t, tl, s: pltpu.make_async_copy(
          t.at[:, pl.ds(st, 128)], s.at[buf], semA).wait())

    def tail():
      eachslab(lambda t, tl, s: pltpu.make_async_copy(
          tl, s.at[buf], semA).wait())
    pl.when(b < NBLK - 1)(full)
    pl.when(b >= NBLK - 1)(tail)

  fire(b0, 0)

  def pair(p, carry):
    f = list(carry)
    for half in range(2):
      j = p * 2 + half
      active = j < BPW2
      b = b0 + j

      def dma_step():
        def fire_next():
          fire(b + 1, 1 - half)
        pl.when(j < BPW2 - 1)(fire_next)
        wait4(b, half)
      pl.when(active)(dma_step)
      su = 2 * half
      f[su] = _process_side(joff + j, active, pbU, sGU.at[half],
                            sMU.at[half], hitU, stage, pidx, outU, semS,
                            su, f[su])
      f[su + 1] = _process_side(joff + j, active, pbI, sGI.at[half],
                                sMI.at[half], hitI, stage, pidx, outI, semS,
                                su + 1, f[su + 1])
    return tuple(f)

  zero = jnp.int32(0)
  fin = lax.fori_loop(0, (BPW2 + 1) // 2, pair, (zero, zero, zero, zero))
  for k in range(4):
    _drain_n(fin[k], stage, outU, semS)


@functools.cache
def _k2():
  return pl.kernel(
      _k2_body,
      out_type=(
          jax.ShapeDtypeStruct((BDUM, OUTP), jnp.float32),
          jax.ShapeDtypeStruct((BDUM, OUTP), jnp.float32),
      ),
      mesh=plsc.VectorSubcoreMesh(core_axis_name="c", subcore_axis_name="s"),
      compiler_params=pltpu.CompilerParams(needs_layout_passes=False),
      scratch_types=[
          pltpu.VMEM((NW, PBRD), jnp.int32),
          pltpu.VMEM((NW, PBRD), jnp.int32),
          pltpu.VMEM((2, F, 128), jnp.float32),
          pltpu.VMEM((2, F, 128), jnp.float32),
          pltpu.VMEM((2, H, 128), jnp.float32),
          pltpu.VMEM((2, H, 128), jnp.float32),
          pltpu.VMEM((HCAP,), jnp.int32),
          pltpu.VMEM((HCAP,), jnp.int32),
          pltpu.VMEM((4, NGMAX * 16, OUTP), jnp.float32),
          pltpu.VMEM((4, NGMAX, 16), jnp.int32),
          pltpu.SemaphoreType.DMA,
          pltpu.SemaphoreType.DMA,
      ],
  )


def _sigmoid(x):
  return 1.0 / (1.0 + jnp.exp(-x))


BLK = 2048


def _tc_dense_body(outu, outi, w1a, w1b, w2, w3, w4, b1, b2, b3, b4,
                   ow, ob, out):
  pu = outu[...]
  pi = outi[...]
  gu = pu[:, :F]
  mu = pu[:, F:F + H]
  gi = pi[:, :F]
  mi = pi[:, F:F + H]
  gmf = _sigmoid(jnp.sum(gu * gi, axis=1, keepdims=True))
  v = jnp.maximum(
      jnp.dot(mu, w1a[...], preferred_element_type=jnp.float32)
      + jnp.dot(mi, w1b[...], preferred_element_type=jnp.float32)
      + b1[...], 0.0)
  v = jnp.maximum(
      jnp.dot(v, w2[...], preferred_element_type=jnp.float32) + b2[...], 0.0)
  v = jnp.maximum(
      jnp.dot(v, w3[...], preferred_element_type=jnp.float32) + b3[...], 0.0)
  mlp = _sigmoid(jnp.sum(v * w4[...], axis=1, keepdims=True) + b4[...])
  oww = ow[...]
  out[...] = _sigmoid(gmf * oww[0:1, 0:1] + mlp * oww[0:1, 1:2] + ob[...])


def _tc_dense(outu, outi, w1a, w1b, w2, w3, w4, b1, b2, b3, b4, ow, ob):
  full = lambda shape: pl.BlockSpec(shape, lambda i: (0, 0))
  return pl.pallas_call(
      _tc_dense_body,
      grid=(B // BLK,),
      in_specs=[
          pl.BlockSpec((BLK, OUTP), lambda i: (i, 0)),
          pl.BlockSpec((BLK, OUTP), lambda i: (i, 0)),
          full((H, F)),
          full((H, F)),
          full((F, F)),
          full((F, F)),
          full((1, F)),
          full((1, F)),
          full((1, F)),
          full((1, F)),
          full((1, 1)),
          full((1, 2)),
          full((1, 1)),
      ],
      out_specs=pl.BlockSpec((BLK, 1), lambda i: (i, 0)),
      out_shape=jax.ShapeDtypeStruct((B, 1), jnp.float32),
  )(outu, outi, w1a, w1b, w2, w3, w4, b1, b2, b3, b4, ow, ob)


@jax.jit
def kernel(user_ids, item_ids, gmf_user_emb, gmf_item_emb, mlp_user_emb,
           mlp_item_emb, fc_w1, fc_b1, fc_w2, fc_b2, fc_w3, fc_b3,
           mlp_out_w, mlp_out_b, out_w, out_b):
  uids = jnp.asarray(user_ids, jnp.int32)
  iids = jnp.asarray(item_ids, jnp.int32)
  posbuf = _k1()(uids, iids)
  pad = lambda x: jnp.pad(x[TAILST:].T, ((0, 0), (0, 128 - (R - TAILST))))
  outU, outI = _k2()(gmf_user_emb.T, gmf_item_emb.T, mlp_user_emb.T,
                     mlp_item_emb.T, posbuf,
                     pad(gmf_user_emb), pad(gmf_item_emb),
                     pad(mlp_user_emb), pad(mlp_item_emb))
  w1a = fc_w1[:, :H].T
  w1b = fc_w1[:, H:].T
  return _tc_dense(outU, outI, w1a, w1b, fc_w2.T, fc_w3.T,
                   mlp_out_w.reshape(1, F), fc_b1.reshape(1, F),
                   fc_b2.reshape(1, F), fc_b3.reshape(1, F),
                   mlp_out_b.reshape(1, 1), out_w, out_b.reshape(1, 1))


# speedup vs baseline: 1.7509x; 1.7138x over previous
import functools

import jax
import jax.numpy as jnp
from jax import lax
from jax.experimental import pallas as pl
from jax.experimental.pallas import tpu as pltpu
from jax.experimental.pallas import tpu_sc as plsc

B = 16384
F = 64
H = 32
NC = 2
NS = 16
NW = NC * NS
BPW = B // NW
CHUNK = 128
NCH = BPW // CHUNK


def _sc_gather_body(gu_t, gi_t, mu_t, mi_t, uidx, iidx,
                    gu_o, gi_o, mu_o, mi_o,
                    uv, iv, gu_v, gi_v, mu_v, mi_v, sem):
  wid = lax.axis_index("s") * NC + lax.axis_index("c")
  base = wid * BPW
  pltpu.sync_copy(uidx.at[wid], uv)
  pltpu.sync_copy(iidx.at[wid], iv)
  copies = []
  for j in range(NCH):
    sl = pl.ds(j * CHUNK, CHUNK)
    copies.append(pltpu.async_copy(gu_t.at[uv.at[j]], gu_v.at[sl], sem))
    copies.append(pltpu.async_copy(gi_t.at[iv.at[j]], gi_v.at[sl], sem))
    copies.append(pltpu.async_copy(mu_t.at[uv.at[j]], mu_v.at[sl], sem))
    copies.append(pltpu.async_copy(mi_t.at[iv.at[j]], mi_v.at[sl], sem))
  for c in copies:
    c.wait()
  pltpu.sync_copy(gu_v, gu_o.at[pl.ds(base, BPW)])
  pltpu.sync_copy(gi_v, gi_o.at[pl.ds(base, BPW)])
  pltpu.sync_copy(mu_v, mu_o.at[pl.ds(base, BPW)])
  pltpu.sync_copy(mi_v, mi_o.at[pl.ds(base, BPW)])


@functools.cache
def _sc_gather():
  return pl.kernel(
      _sc_gather_body,
      out_type=(
          jax.ShapeDtypeStruct((B, F), jnp.float32),
          jax.ShapeDtypeStruct((B, F), jnp.float32),
          jax.ShapeDtypeStruct((B, H), jnp.float32),
          jax.ShapeDtypeStruct((B, H), jnp.float32),
      ),
      mesh=plsc.VectorSubcoreMesh(core_axis_name="c", subcore_axis_name="s"),
      compiler_params=pltpu.CompilerParams(use_tc_tiling_on_sc=False),
      scratch_types=[
          pltpu.VMEM((NCH, CHUNK), jnp.int32),
          pltpu.VMEM((NCH, CHUNK), jnp.int32),
          pltpu.VMEM((BPW, F), jnp.float32),
          pltpu.VMEM((BPW, F), jnp.float32),
          pltpu.VMEM((BPW, H), jnp.float32),
          pltpu.VMEM((BPW, H), jnp.float32),
          pltpu.SemaphoreType.DMA,
      ],
  )


def _sigmoid(x):
  return 1.0 / (1.0 + jnp.exp(-x))


BLK = 2048


def _tc_dense_body(gu, gi, mu, mi, w1a, w1b, w2, w3, w4, b1, b2, b3, b4,
                   ow, ob, out):
  gmf = _sigmoid(jnp.sum(gu[...] * gi[...], axis=1, keepdims=True))
  v = jnp.maximum(
      jnp.dot(mu[...], w1a[...], preferred_element_type=jnp.float32)
      + jnp.dot(mi[...], w1b[...], preferred_element_type=jnp.float32)
      + b1[...], 0.0)
  v = jnp.maximum(
      jnp.dot(v, w2[...], preferred_element_type=jnp.float32) + b2[...], 0.0)
  v = jnp.maximum(
      jnp.dot(v, w3[...], preferred_element_type=jnp.float32) + b3[...], 0.0)
  mlp = _sigmoid(jnp.sum(v * w4[...], axis=1, keepdims=True) + b4[...])
  oww = ow[...]
  out[...] = _sigmoid(gmf * oww[0:1, 0:1] + mlp * oww[0:1, 1:2] + ob[...])


def _tc_dense(gu, gi, mu, mi, w1a, w1b, w2, w3, w4, b1, b2, b3, b4, ow, ob):
  full = lambda shape: pl.BlockSpec(shape, lambda i: (0, 0))
  return pl.pallas_call(
      _tc_dense_body,
      grid=(B // BLK,),
      in_specs=[
          pl.BlockSpec((BLK, F), lambda i: (i, 0)),
          pl.BlockSpec((BLK, F), lambda i: (i, 0)),
          pl.BlockSpec((BLK, H), lambda i: (i, 0)),
          pl.BlockSpec((BLK, H), lambda i: (i, 0)),
          full((H, F)),
          full((H, F)),
          full((F, F)),
          full((F, F)),
          full((1, F)),
          full((1, F)),
          full((1, F)),
          full((1, F)),
          full((1, 1)),
          full((1, 2)),
          full((1, 1)),
      ],
      out_specs=pl.BlockSpec((BLK, 1), lambda i: (i, 0)),
      out_shape=jax.ShapeDtypeStruct((B, 1), jnp.float32),
  )(gu, gi, mu, mi, w1a, w1b, w2, w3, w4, b1, b2, b3, b4, ow, ob)


@jax.jit
def kernel(user_ids, item_ids, gmf_user_emb, gmf_item_emb, mlp_user_emb,
           mlp_item_emb, fc_w1, fc_b1, fc_w2, fc_b2, fc_w3, fc_b3,
           mlp_out_w, mlp_out_b, out_w, out_b):
  uidx = jnp.asarray(user_ids, jnp.int32).reshape(NW, NCH, CHUNK)
  iidx = jnp.asarray(item_ids, jnp.int32).reshape(NW, NCH, CHUNK)
  gu, gi, mu, mi = _sc_gather()(gmf_user_emb, gmf_item_emb, mlp_user_emb,
                                mlp_item_emb, uidx, iidx)
  w1a = fc_w1[:, :H].T
  w1b = fc_w1[:, H:].T
  w2 = fc_w2.T
  w3 = fc_w3.T
  w4 = mlp_out_w.reshape(1, F)
  b1 = fc_b1.reshape(1, F)
  b2 = fc_b2.reshape(1, F)
  b3 = fc_b3.reshape(1, F)
  b4 = mlp_out_b.reshape(1, 1)
  ob = out_b.reshape(1, 1)
  return _tc_dense(gu, gi, mu, mi, w1a, w1b, w2, w3, w4, b1, b2, b3, b4,
                   out_w, ob)
